# Initial kernel scaffold; baseline (speedup 1.0000x reference)
import jax
import jax.numpy as jnp
from jax.experimental import pallas as pl


def kernel(x, edge_index_0, edge_index_1, W_l0, W_r0, b0, W_l1, W_r1, b1):
    raise NotImplementedError("write your pallas kernel here")


# speedup vs baseline: 4.2891x; 4.2891x over previous
import functools

import jax
import jax.numpy as jnp
from jax import lax
from jax.experimental import pallas as pl
from jax.experimental.pallas import tpu as pltpu
from jax.experimental.pallas import tpu_sc as plsc

N1 = 5000
N2 = 1024
D = 128
E0 = 160000
E1 = 32768
NC, NS = 2, 16
NW = NC * NS
CH = 128

L0_CHUNKS = -(-E0 // (NW * CH))
L0_PAD = NW * CH * L0_CHUNKS - E0
L1_CHUNKS = E1 // (NW * CH)


def _make_sc_agg(n_chunks: int, acc_rows: int):
    rows_per_tile = acc_rows // NS
    assert rows_per_tile % 16 == 0
    mesh = plsc.VectorSubcoreMesh(
        core_axis_name="c", subcore_axis_name="s",
        num_cores=NC, num_subcores=NS)

    @functools.partial(
        pl.kernel, mesh=mesh,
        out_type=(jax.ShapeDtypeStruct((NC, N2, D), jnp.float32),
                  jax.ShapeDtypeStruct((NC, N2), jnp.float32)),
        scratch_types=[
            pltpu.VMEM((n_chunks, CH), jnp.int32),
            pltpu.VMEM((n_chunks, CH), jnp.int32),
            pltpu.VMEM((CH, D), jnp.float32),
            pltpu.VMEM((CH,), jnp.float32),
            pltpu.VMEM((CH,), jnp.float32),
            pltpu.VMEM((16, 128), jnp.float32),
            pltpu.VMEM_SHARED((acc_rows, D), jnp.float32),
            pltpu.VMEM_SHARED((2048,), jnp.float32),
            pltpu.SemaphoreType.DMA,
        ])
    def sc_agg(table, src_idx, dst_idx, sum_out, cnt_out,
               idx_s, idx_d, rows, ones_b, zrow, zbuf, acc, cnt_sh, sem):
        c = lax.axis_index("c")
        s = lax.axis_index("s")
        wid = c * NS + s

        zero16 = jnp.zeros((16,), jnp.float32)
        ones16 = jnp.ones((16,), jnp.float32)
        for k in range(CH // 16):
            ones_b[pl.ds(k * 16, 16)] = ones16
            zrow[pl.ds(k * 16, 16)] = zero16
        for r in range(16):
            for l in range(8):
                zbuf[r, pl.ds(l * 16, 16)] = zero16
        for b in range(rows_per_tile // 16):
            pltpu.sync_copy(zbuf,
                            acc.at[pl.ds(s * rows_per_tile + b * 16, 16)])
        pltpu.sync_copy(zrow, cnt_sh.at[pl.ds(s * 128, 128)])

        pltpu.sync_copy(src_idx.at[wid], idx_s)
        pltpu.sync_copy(dst_idx.at[wid], idx_d)
        plsc.subcore_barrier()

        def body(j, carry):
            pltpu.async_copy(table.at[idx_s.at[j]], rows, sem).wait()
            pltpu.sync_copy(rows, acc.at[idx_d.at[j]], add=True)
            pltpu.sync_copy(ones_b, cnt_sh.at[idx_d.at[j]], add=True)
            return carry

        lax.fori_loop(0, n_chunks, body, 0)

        plsc.subcore_barrier()
        nr = N2 // NS
        pltpu.sync_copy(acc.at[pl.ds(s * nr, nr)], rows.at[pl.ds(0, nr)])
        pltpu.sync_copy(rows.at[pl.ds(0, nr)],
                        sum_out.at[c, pl.ds(s * nr, nr)])
        pltpu.sync_copy(cnt_sh.at[pl.ds(s * nr, nr)], zrow.at[pl.ds(0, nr)])
        pltpu.sync_copy(zrow.at[pl.ds(0, nr)],
                        cnt_out.at[c, pl.ds(s * nr, nr)])

    return sc_agg


_SC_AGG_L0 = _make_sc_agg(L0_CHUNKS, 1280)
_SC_AGG_L1 = _make_sc_agg(L1_CHUNKS, N2)


def _dense(sums, cnts, xt, wl, wr, b, do_relu: bool):
    def body(s_ref, c_ref, x_ref, wl_ref, wr_ref, b_ref, o_ref):
        sm = s_ref[0] + s_ref[1]
        cn = c_ref[0] + c_ref[1]
        agg = sm / jnp.maximum(cn, 1.0)
        r = (jnp.dot(agg, wl_ref[...], preferred_element_type=jnp.float32)
             + jnp.dot(x_ref[...], wr_ref[...],
                       preferred_element_type=jnp.float32)
             + b_ref[...])
        o_ref[...] = jnp.maximum(r, 0.0) if do_relu else r

    return pl.pallas_call(
        body,
        out_shape=jax.ShapeDtypeStruct((N2, D), jnp.float32),
    )(sums, cnts, xt, wl, wr, b)


def kernel(x, edge_index_0, edge_index_1, W_l0, W_r0, b0, W_l1, W_r1, b1):
    src0 = edge_index_0[0].astype(jnp.int32)
    dst0 = edge_index_0[1].astype(jnp.int32)
    dst0 = jnp.where(dst0 < N2, dst0, N2)
    src0 = jnp.concatenate([src0, jnp.zeros((L0_PAD,), jnp.int32)])
    dst0 = jnp.concatenate([dst0, jnp.full((L0_PAD,), N2, jnp.int32)])
    src0 = src0.reshape(NW, L0_CHUNKS, CH)
    dst0 = dst0.reshape(NW, L0_CHUNKS, CH)

    sum0, cnt0 = _SC_AGG_L0(x, src0, dst0)
    xt0 = lax.slice(x, (0, 0), (N2, D))
    h = _dense(sum0, cnt0[..., None], xt0,
               W_l0, W_r0, b0.reshape(1, D), True)

    src1 = edge_index_1[0].astype(jnp.int32).reshape(NW, L1_CHUNKS, CH)
    dst1 = edge_index_1[1].astype(jnp.int32).reshape(NW, L1_CHUNKS, CH)
    sum1, cnt1 = _SC_AGG_L1(h, src1, dst1)
    out = _dense(sum1, cnt1[..., None], h,
                 W_l1, W_r1, b1.reshape(1, D), False)
    return out


# speedup vs baseline: 7.5062x; 1.7501x over previous
import functools

import jax
import jax.numpy as jnp
from jax import lax
from jax.experimental import pallas as pl
from jax.experimental.pallas import tpu as pltpu
from jax.experimental.pallas import tpu_sc as plsc

N1 = 5000
N2 = 1024
D = 128
E0 = 160000
E1 = 32768
NC, NS = 2, 16
NW = NC * NS
CH = 128

L0_CHUNKS = -(-E0 // (NW * CH))
L0_PAD = NW * CH * L0_CHUNKS - E0
L1_CHUNKS = E1 // (NW * CH)


def _make_sc_agg(n_chunks: int, acc_rows: int):
    rows_per_tile = acc_rows // NS
    assert rows_per_tile % 16 == 0
    mesh = plsc.VectorSubcoreMesh(
        core_axis_name="c", subcore_axis_name="s",
        num_cores=NC, num_subcores=NS)

    @functools.partial(
        pl.kernel, mesh=mesh,
        out_type=(jax.ShapeDtypeStruct((NC, N2, D), jnp.float32),
                  jax.ShapeDtypeStruct((NC, N2), jnp.float32)),
        scratch_types=[
            pltpu.VMEM((n_chunks, CH), jnp.int32),
            pltpu.VMEM((n_chunks, CH), jnp.int32),
            pltpu.VMEM((CH, D), jnp.float32),
            pltpu.VMEM((CH,), jnp.float32),
            pltpu.VMEM((CH,), jnp.float32),
            pltpu.VMEM((16, 128), jnp.float32),
            pltpu.VMEM_SHARED((acc_rows, D), jnp.float32),
            pltpu.VMEM_SHARED((2048,), jnp.float32),
            pltpu.SemaphoreType.DMA,
        ])
    def sc_agg(table, src_idx, dst_idx, sum_out, cnt_out,
               idx_s, idx_d, rows, ones_b, zrow, zbuf, acc, cnt_sh, sem):
        c = lax.axis_index("c")
        s = lax.axis_index("s")
        wid = c * NS + s

        zero16 = jnp.zeros((16,), jnp.float32)
        ones16 = jnp.ones((16,), jnp.float32)
        for k in range(CH // 16):
            ones_b[pl.ds(k * 16, 16)] = ones16
            zrow[pl.ds(k * 16, 16)] = zero16
        for r in range(16):
            for l in range(8):
                zbuf[r, pl.ds(l * 16, 16)] = zero16
        for b in range(rows_per_tile // 16):
            pltpu.sync_copy(zbuf,
                            acc.at[pl.ds(s * rows_per_tile + b * 16, 16)])
        pltpu.sync_copy(zrow, cnt_sh.at[pl.ds(s * 128, 128)])

        pltpu.sync_copy(src_idx.at[wid], idx_s)
        pltpu.sync_copy(dst_idx.at[wid], idx_d)
        plsc.subcore_barrier()

        def body(j, carry):
            pltpu.async_copy(table.at[idx_s.at[j]], rows, sem).wait()
            pltpu.sync_copy(rows, acc.at[idx_d.at[j]], add=True)
            pltpu.sync_copy(ones_b, cnt_sh.at[idx_d.at[j]], add=True)
            return carry

        lax.fori_loop(0, n_chunks, body, 0)

        plsc.subcore_barrier()
        nr = N2 // NS
        pltpu.sync_copy(acc.at[pl.ds(s * nr, nr)], rows.at[pl.ds(0, nr)])
        pltpu.sync_copy(rows.at[pl.ds(0, nr)],
                        sum_out.at[c, pl.ds(s * nr, nr)])
        pltpu.sync_copy(cnt_sh.at[pl.ds(s * nr, nr)], zrow.at[pl.ds(0, nr)])
        pltpu.sync_copy(zrow.at[pl.ds(0, nr)],
                        cnt_out.at[c, pl.ds(s * nr, nr)])

    return sc_agg


_SC_AGG_L1 = _make_sc_agg(L1_CHUNKS, N2)

EPW = L0_CHUNKS * CH
SRC_BITS = 13


def _make_sc_agg_filtered(acc_rows: int):
    rows_per_tile = acc_rows // NS
    assert rows_per_tile % 16 == 0
    mesh = plsc.VectorSubcoreMesh(
        core_axis_name="c", subcore_axis_name="s",
        num_cores=NC, num_subcores=NS)
    thresh = N2 << SRC_BITS
    smask = (1 << SRC_BITS) - 1

    @functools.partial(
        pl.kernel, mesh=mesh,
        out_type=(jax.ShapeDtypeStruct((NC, N2, D), jnp.float32),
                  jax.ShapeDtypeStruct((NC, N2), jnp.float32)),
        scratch_types=[
            pltpu.VMEM((EPW,), jnp.int32),
            pltpu.VMEM((EPW + CH,), jnp.int32),
            pltpu.VMEM((1, CH), jnp.int32),
            pltpu.VMEM((1, CH), jnp.int32),
            pltpu.VMEM((CH, D), jnp.float32),
            pltpu.VMEM((CH,), jnp.float32),
            pltpu.VMEM((CH,), jnp.float32),
            pltpu.VMEM((16, 128), jnp.float32),
            pltpu.VMEM_SHARED((acc_rows, D), jnp.float32),
            pltpu.VMEM_SHARED((2048,), jnp.float32),
            pltpu.SemaphoreType.DMA,
        ])
    def sc_agg_f(table, edges, sum_out, cnt_out,
                 slab, comp, sstage, dstage, rows, ones_b, zrow, zbuf,
                 acc, cnt_sh, sem):
        c = lax.axis_index("c")
        s = lax.axis_index("s")
        wid = c * NS + s

        zero16 = jnp.zeros((16,), jnp.float32)
        ones16 = jnp.ones((16,), jnp.float32)
        for k in range(CH // 16):
            ones_b[pl.ds(k * 16, 16)] = ones16
            zrow[pl.ds(k * 16, 16)] = zero16
        for r in range(16):
            for l in range(8):
                zbuf[r, pl.ds(l * 16, 16)] = zero16
        for b in range(rows_per_tile // 16):
            pltpu.sync_copy(zbuf,
                            acc.at[pl.ds(s * rows_per_tile + b * 16, 16)])
        pltpu.sync_copy(zrow, cnt_sh.at[pl.ds(s * 128, 128)])

        pltpu.sync_copy(edges.at[wid], slab)
        plsc.subcore_barrier()

        one16i = jnp.ones((16,), jnp.int32)

        def compact(t, off):
            pv = slab[pl.ds(t * 16, 16)]
            o = off
            for k in range(16):
                e = pv[k]
                comp[pl.ds(o, 16)] = one16i * e
                o = o + jnp.where(e < thresh, 1, 0).astype(jnp.int32)
            return o

        off = lax.fori_loop(0, EPW // 16, compact, jnp.int32(0))
        pad16 = jnp.full((16,), N2 << SRC_BITS, jnp.int32)
        for t in range(CH // 16):
            comp[pl.ds(off + t * 16, 16)] = pad16
        n_ch = lax.shift_right_logical(off + (CH - 1), 7)

        def body(j, carry):
            for l in range(CH // 16):
                p = comp[pl.ds(j * CH + l * 16, 16)]
                sstage[0, pl.ds(l * 16, 16)] = jnp.bitwise_and(p, smask)
                dstage[0, pl.ds(l * 16, 16)] = lax.shift_right_logical(
                    p, SRC_BITS)
            pltpu.async_copy(table.at[sstage.at[0]], rows, sem).wait()
            pltpu.sync_copy(rows, acc.at[dstage.at[0]], add=True)
            pltpu.sync_copy(ones_b, cnt_sh.at[dstage.at[0]], add=True)
            return carry

        lax.fori_loop(0, n_ch, body, 0)

        plsc.subcore_barrier()
        nr = N2 // NS
        pltpu.sync_copy(acc.at[pl.ds(s * nr, nr)], rows.at[pl.ds(0, nr)])
        pltpu.sync_copy(rows.at[pl.ds(0, nr)],
                        sum_out.at[c, pl.ds(s * nr, nr)])
        pltpu.sync_copy(cnt_sh.at[pl.ds(s * nr, nr)], zrow.at[pl.ds(0, nr)])
        pltpu.sync_copy(zrow.at[pl.ds(0, nr)],
                        cnt_out.at[c, pl.ds(s * nr, nr)])

    return sc_agg_f


_SC_AGG_L0F = _make_sc_agg_filtered(1280)


def _dense(sums, cnts, xt, wl, wr, b, do_relu: bool):
    def body(s_ref, c_ref, x_ref, wl_ref, wr_ref, b_ref, o_ref):
        sm = s_ref[0] + s_ref[1]
        cn = c_ref[0] + c_ref[1]
        agg = sm / jnp.maximum(cn, 1.0)
        r = (jnp.dot(agg, wl_ref[...], preferred_element_type=jnp.float32)
             + jnp.dot(x_ref[...], wr_ref[...],
                       preferred_element_type=jnp.float32)
             + b_ref[...])
        o_ref[...] = jnp.maximum(r, 0.0) if do_relu else r

    return pl.pallas_call(
        body,
        out_shape=jax.ShapeDtypeStruct((N2, D), jnp.float32),
    )(sums, cnts, xt, wl, wr, b)


def kernel(x, edge_index_0, edge_index_1, W_l0, W_r0, b0, W_l1, W_r1, b1):
    src0 = edge_index_0[0].astype(jnp.int32)
    dst0 = edge_index_0[1].astype(jnp.int32)
    dst0 = jnp.where(dst0 < N2, dst0, N2)
    packed0 = jnp.concatenate([
        (dst0 << SRC_BITS) | src0,
        jnp.full((L0_PAD,), N2 << SRC_BITS, jnp.int32),
    ]).reshape(NW, EPW)

    sum0, cnt0 = _SC_AGG_L0F(x, packed0)
    xt0 = lax.slice(x, (0, 0), (N2, D))
    h = _dense(sum0, cnt0[..., None], xt0,
               W_l0, W_r0, b0.reshape(1, D), True)

    src1 = edge_index_1[0].astype(jnp.int32).reshape(NW, L1_CHUNKS, CH)
    dst1 = edge_index_1[1].astype(jnp.int32).reshape(NW, L1_CHUNKS, CH)
    sum1, cnt1 = _SC_AGG_L1(h, src1, dst1)
    out = _dense(sum1, cnt1[..., None], h,
                 W_l1, W_r1, b1.reshape(1, D), False)
    return out
